# Initial kernel scaffold; baseline (speedup 1.0000x reference)
import jax
import jax.numpy as jnp
from jax.experimental import pallas as pl


def kernel(x, edge_index, etype, W_msg, b_msg, gru_w_ih, gru_w_hh, gru_b_ih, gru_b_hh, clf_w, clf_b):
    raise NotImplementedError("write your pallas kernel here")


# speedup vs baseline: 16.0908x; 16.0908x over previous
import functools

import jax
import jax.numpy as jnp
from jax import lax
from jax.experimental import pallas as pl
from jax.experimental.pallas import tpu as pltpu
from jax.experimental.pallas import tpu_sc as plsc

_N = 50000
_E = 800000
_OUT = 64
_HALF = 32
_T = 4
_STEPS = 8
_NCORES = 2
_NTILES = 16
_CHUNK = 128
_KB = 4
_CPT = 392
_G = _CPT // _KB
_NCH = _CPT * _NTILES
_EP = _NCH * _CHUNK
_NP = 50048
_RPT = _NP // _NTILES
_NB = 2000
_GRID = _N // _NB


def _sc_aggregate(ht_flat, gidx2, sdst2, zeros):
    mesh = plsc.VectorSubcoreMesh(
        core_axis_name="c", subcore_axis_name="s",
        num_cores=_NCORES, num_subcores=_NTILES)

    @functools.partial(
        pl.kernel,
        out_type=jax.ShapeDtypeStruct((_NCORES, _NP, _HALF), jnp.float32),
        mesh=mesh,
        scratch_types=[
            pltpu.VMEM((_KB, _CHUNK), jnp.int32),
            pltpu.VMEM((_KB, _CHUNK), jnp.int32),
            pltpu.VMEM((_KB, _CHUNK, _HALF), jnp.float32),
            pltpu.VMEM_SHARED((_NP, _HALF), jnp.float32),
            pltpu.SemaphoreType.DMA,
        ],
        compiler_params=pltpu.CompilerParams(use_tc_tiling_on_sc=False),
    )
    def k(ht_hbm, gidx_hbm, sdst_hbm, zeros_hbm, out_hbm,
          gidx_v, sidx_v, rows_v, acc, sem):
        c = lax.axis_index("c")
        s = lax.axis_index("s")
        r0 = s * _RPT
        pltpu.sync_copy(zeros_hbm.at[pl.ds(r0, _RPT)], acc.at[pl.ds(r0, _RPT)])
        plsc.subcore_barrier()

        def group(g, carry):
            base = s * _CPT + g * _KB
            pltpu.sync_copy(gidx_hbm.at[c, pl.ds(base, _KB)], gidx_v)
            pltpu.sync_copy(sdst_hbm.at[pl.ds(base, _KB)], sidx_v)
            descs = [pltpu.async_copy(ht_hbm.at[gidx_v.at[j]], rows_v.at[j], sem)
                     for j in range(_KB)]
            for j in range(_KB):
                descs[j].wait()
                pltpu.sync_copy(rows_v.at[j], acc.at[sidx_v.at[j]], add=True)
            return carry

        lax.fori_loop(0, _G, group, 0)
        plsc.subcore_barrier()
        pltpu.sync_copy(acc.at[pl.ds(r0, _RPT)], out_hbm.at[c, pl.ds(r0, _RPT)])

    return k(ht_flat, gidx2, sdst2, zeros)


def _tc_prologue(h, WT, b4):
    def body(h_ref, wt_ref, b_ref, out_ref):
        hb = h_ref[...]
        for t in range(_T):
            for c in range(_NCORES):
                out_ref[c, t] = (
                    jnp.dot(hb, wt_ref[t, c], preferred_element_type=jnp.float32)
                    + b_ref[t, c])

    return pl.pallas_call(
        body,
        grid=(_GRID,),
        in_specs=[
            pl.BlockSpec((_NB, _OUT), lambda i: (i, 0)),
            pl.BlockSpec((_T, _NCORES, _OUT, _HALF), lambda i: (0, 0, 0, 0)),
            pl.BlockSpec((_T, _NCORES, _HALF), lambda i: (0, 0, 0)),
        ],
        out_specs=pl.BlockSpec((_NCORES, _T, _NB, _HALF), lambda i: (0, 0, i, 0)),
        out_shape=jax.ShapeDtypeStruct((_NCORES, _T, _N, _HALF), jnp.float32),
    )(h, WT, b4)


def _gru(a, h, wih_ref, whh_ref, bih_ref, bhh_ref):
    gi = jnp.dot(a, wih_ref[...], preferred_element_type=jnp.float32) + bih_ref[...]
    gh = jnp.dot(h, whh_ref[...], preferred_element_type=jnp.float32) + bhh_ref[...]
    r = jax.nn.sigmoid(gi[:, :_OUT] + gh[:, :_OUT])
    z = jax.nn.sigmoid(gi[:, _OUT:2 * _OUT] + gh[:, _OUT:2 * _OUT])
    n = jnp.tanh(gi[:, 2 * _OUT:] + r * gh[:, 2 * _OUT:])
    return (1.0 - z) * n + z * h


def _tc_step(a2, h, wih, whh, bih, bhh, WT, b4):
    def body(a2_ref, h_ref, wih_ref, whh_ref, bih_ref, bhh_ref, wt_ref, b_ref,
             h_out, ht_out):
        a = jnp.concatenate([a2_ref[0], a2_ref[1]], axis=-1)
        hn = _gru(a, h_ref[...], wih_ref, whh_ref, bih_ref, bhh_ref)
        h_out[...] = hn
        for t in range(_T):
            for c in range(_NCORES):
                ht_out[c, t] = (
                    jnp.dot(hn, wt_ref[t, c], preferred_element_type=jnp.float32)
                    + b_ref[t, c])

    return pl.pallas_call(
        body,
        grid=(_GRID,),
        in_specs=[
            pl.BlockSpec((_NCORES, _NB, _HALF), lambda i: (0, i, 0)),
            pl.BlockSpec((_NB, _OUT), lambda i: (i, 0)),
            pl.BlockSpec((_OUT, 3 * _OUT), lambda i: (0, 0)),
            pl.BlockSpec((_OUT, 3 * _OUT), lambda i: (0, 0)),
            pl.BlockSpec((1, 3 * _OUT), lambda i: (0, 0)),
            pl.BlockSpec((1, 3 * _OUT), lambda i: (0, 0)),
            pl.BlockSpec((_T, _NCORES, _OUT, _HALF), lambda i: (0, 0, 0, 0)),
            pl.BlockSpec((_T, _NCORES, _HALF), lambda i: (0, 0, 0)),
        ],
        out_specs=[
            pl.BlockSpec((_NB, _OUT), lambda i: (i, 0)),
            pl.BlockSpec((_NCORES, _T, _NB, _HALF), lambda i: (0, 0, i, 0)),
        ],
        out_shape=[
            jax.ShapeDtypeStruct((_N, _OUT), jnp.float32),
            jax.ShapeDtypeStruct((_NCORES, _T, _N, _HALF), jnp.float32),
        ],
    )(a2, h, wih, whh, bih, bhh, WT, b4)


def _tc_final(a2, h, h1, wih, whh, bih, bhh, clf_wt, clf_b2):
    def body(a2_ref, h_ref, h1_ref, wih_ref, whh_ref, bih_ref, bhh_ref,
             cw_ref, cb_ref, feats_out, res_out):
        i = pl.program_id(0)
        a = jnp.concatenate([a2_ref[0], a2_ref[1]], axis=-1)
        hn = _gru(a, h_ref[...], wih_ref, whh_ref, bih_ref, bhh_ref)
        part = jnp.sum(hn + h1_ref[...], axis=0, keepdims=True)

        @pl.when(i == 0)
        def _():
            feats_out[...] = jnp.zeros_like(feats_out)

        feats_out[...] += part

        @pl.when(i == _GRID - 1)
        def _():
            res_out[...] = (
                jnp.dot(feats_out[...], cw_ref[...],
                        preferred_element_type=jnp.float32) + cb_ref[...])

    return pl.pallas_call(
        body,
        grid=(_GRID,),
        in_specs=[
            pl.BlockSpec((_NCORES, _NB, _HALF), lambda i: (0, i, 0)),
            pl.BlockSpec((_NB, _OUT), lambda i: (i, 0)),
            pl.BlockSpec((_NB, _OUT), lambda i: (i, 0)),
            pl.BlockSpec((_OUT, 3 * _OUT), lambda i: (0, 0)),
            pl.BlockSpec((_OUT, 3 * _OUT), lambda i: (0, 0)),
            pl.BlockSpec((1, 3 * _OUT), lambda i: (0, 0)),
            pl.BlockSpec((1, 3 * _OUT), lambda i: (0, 0)),
            pl.BlockSpec((_OUT, 2), lambda i: (0, 0)),
            pl.BlockSpec((1, 2), lambda i: (0, 0)),
        ],
        out_specs=[
            pl.BlockSpec((1, _OUT), lambda i: (0, 0)),
            pl.BlockSpec((1, 2), lambda i: (0, 0)),
        ],
        out_shape=[
            jax.ShapeDtypeStruct((1, _OUT), jnp.float32),
            jax.ShapeDtypeStruct((1, 2), jnp.float32),
        ],
    )(a2, h, h1, wih, whh, bih, bhh, clf_wt, clf_b2)


def kernel(x, edge_index, etype, W_msg, b_msg, gru_w_ih, gru_w_hh,
           gru_b_ih, gru_b_hh, clf_w, clf_b):
    src = edge_index[0]
    dst = edge_index[1]

    gidx = etype * _N + src
    pad = _EP - _E
    gidx_p = jnp.concatenate([gidx, jnp.zeros((pad,), jnp.int32)])
    dst_p = jnp.concatenate([dst, jnp.full((pad,), _N, jnp.int32)])
    gidx2 = jnp.stack([gidx_p, gidx_p + _T * _N]).reshape(_NCORES, _NCH, _CHUNK)
    sdst2 = dst_p.reshape(_NCH, _CHUNK)
    zeros = jnp.zeros((_NP, _HALF), jnp.float32)

    WT = jnp.transpose(W_msg, (0, 2, 1)).reshape(_T, _OUT, _NCORES, _HALF)
    WT = jnp.transpose(WT, (0, 2, 1, 3))
    b4 = b_msg.reshape(_T, _NCORES, _HALF)
    wih = gru_w_ih.T
    whh = gru_w_hh.T
    bih = gru_b_ih.reshape(1, 3 * _OUT)
    bhh = gru_b_hh.reshape(1, 3 * _OUT)
    clf_wt = clf_w.T
    clf_b2 = clf_b.reshape(1, 2)

    h1 = x
    if x.shape[1] < _OUT:
        h1 = jnp.concatenate(
            [x, jnp.zeros((x.shape[0], _OUT - x.shape[1]), x.dtype)], axis=-1)
    h = h1

    ht2 = _tc_prologue(h, WT, b4)
    feats = None
    for s_i in range(_STEPS):
        a2 = _sc_aggregate(ht2.reshape(_NCORES * _T * _N, _HALF),
                           gidx2, sdst2, zeros)
        if s_i < _STEPS - 1:
            h, ht2 = _tc_step(a2, h, wih, whh, bih, bhh, WT, b4)
        else:
            feats, res = _tc_final(a2, h, h1, wih, whh, bih, bhh,
                                   clf_wt, clf_b2)
    return res[0]
